# baseline (speedup 1.0000x reference)
import functools

import jax
import jax.numpy as jnp
from jax import lax
from jax.experimental import pallas as pl
from jax.experimental.pallas import tpu as pltpu
from jax.experimental.pallas import tpu_sc as plsc

_VOCAB = 50
_HIDDEN = 32
_BATCH = 4096
_SEQ = 200
_N = _BATCH * _SEQ

_TPAD = 65
_NC = 2
_NS = 16
_NW = _NC * _NS
_B_PER_W = _BATCH // _NW
_NB = 4
_TOK = _NB * _SEQ
_GROUPS = _TOK // 16
_N_CHUNKS = _B_PER_W // _NB


def _fuse_table_body(e_ref, w_ref, b_ref, t_ref):
    t_ref[...] = (
        jnp.dot(e_ref[...], w_ref[...], preferred_element_type=jnp.float32)
        + b_ref[...]
    )


def _fuse_table(embed_table, W, b):
    wp = jnp.zeros((_HIDDEN, _TPAD), jnp.float32).at[:, :_VOCAB].set(W)
    bp = jnp.zeros((1, _TPAD), jnp.float32).at[0, :_VOCAB].set(b)
    t = pl.pallas_call(
        _fuse_table_body,
        out_shape=jax.ShapeDtypeStruct((_VOCAB, _TPAD), jnp.float32),
    )(embed_table, wp, bp)
    return t.reshape(_VOCAB * _TPAD)


def _gather_body(t_hbm, idx_hbm, out_hbm,
                 table_v, idx_v0, idx_v1, out_v0, out_v1,
                 sem_t, sem_i0, sem_i1, sem_o0, sem_o1):
    wid = lax.axis_index("s") * _NC + lax.axis_index("c")
    base_b = wid * _B_PER_W

    pltpu.sync_copy(t_hbm, table_v)

    lane = lax.iota(jnp.int32, 16)
    idx_bufs = (idx_v0, idx_v1)
    out_bufs = (out_v0, out_v1)
    idx_sems = (sem_i0, sem_i1)
    out_sems = (sem_o0, sem_o1)

    def start_idx(c, buf, sem):
        b0 = base_b + c * _NB
        for j in range(_NB):
            pltpu.async_copy(
                idx_hbm.at[b0 + j], buf.at[pl.ds(j * _SEQ, _SEQ)], sem
            )

    def wait_idx(c, buf, sem):
        b0 = base_b + c * _NB
        for j in range(_NB):
            pltpu.make_async_copy(
                idx_hbm.at[b0 + j], buf.at[pl.ds(j * _SEQ, _SEQ)], sem
            ).wait()

    def compute(idx_v, out_v):
        def group_body(g, _):
            tt = g * 16 + lane
            jb = tt // _SEQ
            s = tt - jb * _SEQ
            rowids = idx_v[pl.ds(g * 16, 16)]
            fb = rowids * _TPAD
            for col in range(_VOCAB):
                vals = plsc.load_gather(table_v, [fb + col])
                colv = jnp.full((16,), col, jnp.int32)
                plsc.store_scatter(out_v, [jb, s, colv], vals)
            return 0

        lax.fori_loop(0, _GROUPS, group_body, 0)

    start_idx(0, idx_bufs[0], idx_sems[0])

    def pair_body(cc, _):
        for bsel in range(2):
            c = cc * 2 + bsel
            b0 = base_b + c * _NB
            ivb, isem = idx_bufs[bsel], idx_sems[bsel]
            ovb, osem = out_bufs[bsel], out_sems[bsel]
            wait_idx(c, ivb, isem)

            @pl.when(c + 1 < _N_CHUNKS)
            def _():
                start_idx(c + 1, idx_bufs[1 - bsel], idx_sems[1 - bsel])

            @pl.when(c >= 2)
            def _():
                pltpu.make_async_copy(
                    ovb, out_hbm.at[pl.ds(b0, _NB)], osem
                ).wait()

            compute(ivb, ovb)
            pltpu.async_copy(ovb, out_hbm.at[pl.ds(b0, _NB)], osem)
        return 0

    lax.fori_loop(0, _N_CHUNKS // 2, pair_body, 0)

    for bsel in range(2):
        c = _N_CHUNKS - 2 + bsel
        b0 = base_b + c * _NB
        pltpu.make_async_copy(
            out_bufs[bsel], out_hbm.at[pl.ds(b0, _NB)], out_sems[bsel]
        ).wait()


def _sc_gather(table_flat, idx2d):
    mesh = plsc.VectorSubcoreMesh(core_axis_name="c", subcore_axis_name="s")
    kern = functools.partial(
        pl.kernel,
        mesh=mesh,
        compiler_params=pltpu.CompilerParams(
            needs_layout_passes=False, use_tc_tiling_on_sc=False
        ),
        out_type=jax.ShapeDtypeStruct((_BATCH, _SEQ, _VOCAB), jnp.float32),
        scratch_types=[
            pltpu.VMEM((_VOCAB * _TPAD,), jnp.float32),
            pltpu.VMEM((_TOK,), jnp.int32),
            pltpu.VMEM((_TOK,), jnp.int32),
            pltpu.VMEM((_NB, _SEQ, _VOCAB), jnp.float32),
            pltpu.VMEM((_NB, _SEQ, _VOCAB), jnp.float32),
            pltpu.SemaphoreType.DMA,
            pltpu.SemaphoreType.DMA,
            pltpu.SemaphoreType.DMA,
            pltpu.SemaphoreType.DMA,
            pltpu.SemaphoreType.DMA,
        ],
    )(_gather_body)
    return kern(table_flat, idx2d)


def kernel(indices, embed_table, W, b):
    table_flat = _fuse_table(embed_table, W, b)
    return _sc_gather(table_flat, indices)


# speedup vs baseline: 1.3301x; 1.3301x over previous
import functools

import jax
import jax.numpy as jnp
from jax import lax
from jax.experimental import pallas as pl
from jax.experimental.pallas import tpu as pltpu
from jax.experimental.pallas import tpu_sc as plsc

_VOCAB = 50
_HIDDEN = 32
_BATCH = 4096
_SEQ = 200

_TPAD = 65
_NC = 2
_NS = 16
_NW = _NC * _NS
_B_PER_W = _BATCH // _NW
_NB = 2
_N_CHUNKS = _B_PER_W // _NB
_SGROUPS = 13
_COLS = (0, 16, 32, 34)

_BCAST_DNUMS = lax.GatherDimensionNumbers(
    offset_dims=(), collapsed_slice_dims=(0,), start_index_map=(0,)
)


def _bcast_lane(vec, j):
    return lax.gather(
        vec,
        jnp.full((16, 1), j, jnp.int32),
        _BCAST_DNUMS,
        (1,),
        mode=lax.GatherScatterMode.PROMISE_IN_BOUNDS,
    )


def _fuse_table_body(e_ref, w_ref, b_ref, t_ref):
    t_ref[...] = (
        jnp.dot(e_ref[...], w_ref[...], preferred_element_type=jnp.float32)
        + b_ref[...]
    )


def _fuse_table(embed_table, W, b):
    wp = jnp.zeros((_HIDDEN, _TPAD), jnp.float32).at[:, :_VOCAB].set(W)
    bp = jnp.zeros((1, _TPAD), jnp.float32).at[0, :_VOCAB].set(b)
    t = pl.pallas_call(
        _fuse_table_body,
        out_shape=jax.ShapeDtypeStruct((_VOCAB, _TPAD), jnp.float32),
    )(embed_table, wp, bp)
    return t.reshape(_VOCAB * _TPAD)


def _gather_body(t_hbm, idx_hbm, out_hbm,
                 table_v, idx_v0, idx_v1, out_v0, out_v1,
                 sem_i0, sem_i1, sem_o0, sem_o1):
    wid = lax.axis_index("s") * _NC + lax.axis_index("c")
    base_b = wid * _B_PER_W

    pltpu.sync_copy(t_hbm, table_v)

    lane = lax.iota(jnp.int32, 16)
    colio = [lane + c0 for c0 in _COLS]
    idx_bufs = (idx_v0, idx_v1)
    out_bufs = (out_v0, out_v1)
    idx_sems = (sem_i0, sem_i1)
    out_sems = (sem_o0, sem_o1)

    def idx_copy(c, buf, sem):
        b0 = base_b + c * _NB
        return pltpu.make_async_copy(idx_hbm.at[pl.ds(b0, _NB)], buf, sem)

    def compute(idx_v, out_v):
        def group_body(g, _):
            s0 = jnp.where(g == _SGROUPS - 1, _SEQ - 16, g * 16)
            for jb in range(_NB):
                idxvec = idx_v[jb, pl.ds(s0, 16)]
                fball = idxvec * _TPAD
                for j in range(16):
                    fbb = _bcast_lane(fball, j)
                    for k in range(len(_COLS)):
                        vals = plsc.load_gather(table_v, [fbb + colio[k]])
                        out_v[jb, s0 + j, pl.ds(_COLS[k], 16)] = vals
            return 0

        lax.fori_loop(0, _SGROUPS, group_body, 0)

    idx_copy(0, idx_bufs[0], idx_sems[0]).start()

    def pair_body(cc, _):
        for bsel in range(2):
            c = cc * 2 + bsel
            b0 = base_b + c * _NB
            ivb, isem = idx_bufs[bsel], idx_sems[bsel]
            ovb, osem = out_bufs[bsel], out_sems[bsel]
            idx_copy(c, ivb, isem).wait()

            @pl.when(c + 1 < _N_CHUNKS)
            def _():
                idx_copy(c + 1, idx_bufs[1 - bsel], idx_sems[1 - bsel]).start()

            @pl.when(c >= 2)
            def _():
                pltpu.make_async_copy(
                    ovb, out_hbm.at[pl.ds(b0, _NB)], osem
                ).wait()

            compute(ivb, ovb)
            pltpu.async_copy(ovb, out_hbm.at[pl.ds(b0, _NB)], osem)
        return 0

    lax.fori_loop(0, _N_CHUNKS // 2, pair_body, 0)

    for bsel in range(2):
        c = _N_CHUNKS - 2 + bsel
        b0 = base_b + c * _NB
        pltpu.make_async_copy(
            out_bufs[bsel], out_hbm.at[pl.ds(b0, _NB)], out_sems[bsel]
        ).wait()


def _sc_gather(table_flat, idx2d):
    mesh = plsc.VectorSubcoreMesh(core_axis_name="c", subcore_axis_name="s")
    kern = functools.partial(
        pl.kernel,
        mesh=mesh,
        compiler_params=pltpu.CompilerParams(needs_layout_passes=False),
        out_type=jax.ShapeDtypeStruct((_BATCH, _SEQ, _VOCAB), jnp.float32),
        scratch_types=[
            pltpu.VMEM((_VOCAB * _TPAD,), jnp.float32),
            pltpu.VMEM((_NB, _SEQ), jnp.int32),
            pltpu.VMEM((_NB, _SEQ), jnp.int32),
            pltpu.VMEM((_NB, _SEQ, _VOCAB), jnp.float32),
            pltpu.VMEM((_NB, _SEQ, _VOCAB), jnp.float32),
            pltpu.SemaphoreType.DMA,
            pltpu.SemaphoreType.DMA,
            pltpu.SemaphoreType.DMA,
            pltpu.SemaphoreType.DMA,
        ],
    )(_gather_body)
    return kern(table_flat, idx2d)


def kernel(indices, embed_table, W, b):
    table_flat = _fuse_table(embed_table, W, b)
    return _sc_gather(table_flat, indices)


# speedup vs baseline: 2.3224x; 1.7461x over previous
import functools

import jax
import jax.numpy as jnp
from jax import lax
from jax.experimental import pallas as pl
from jax.experimental.pallas import tpu as pltpu
from jax.experimental.pallas import tpu_sc as plsc

_VOCAB = 50
_HIDDEN = 32
_BATCH = 4096
_SEQ = 200

_TPAD = 65
_NC = 2
_NS = 16
_NW = _NC * _NS
_BBLK = _BATCH // _NW
_NSQ = 4
_N_CHUNKS = _SEQ // _NSQ


def _fuse_body(e_ref, w_ref, b_ref, idx_ref, t_ref, it_ref):
    t_ref[...] = (
        jnp.dot(e_ref[...], w_ref[...], preferred_element_type=jnp.float32)
        + b_ref[...]
    )
    it_ref[...] = idx_ref[...].T


def _fuse_table(embed_table, W, b, indices):
    wp = jnp.zeros((_HIDDEN, _TPAD), jnp.float32).at[:, :_VOCAB].set(W)
    bp = jnp.zeros((1, _TPAD), jnp.float32).at[0, :_VOCAB].set(b)
    t, idx_t = pl.pallas_call(
        _fuse_body,
        out_shape=(
            jax.ShapeDtypeStruct((_VOCAB, _TPAD), jnp.float32),
            jax.ShapeDtypeStruct((_SEQ, _BATCH), jnp.int32),
        ),
    )(embed_table, wp, bp, indices)
    return t.reshape(_VOCAB * _TPAD), idx_t


def _gather_body(t_hbm, idx_hbm, out_hbm,
                 table_v, idx_v0, idx_v1, out_v0, out_v1,
                 sem_i0, sem_i1, sem_o0, sem_o1):
    wid = lax.axis_index("s") * _NC + lax.axis_index("c")
    bw = wid * _BBLK

    pltpu.sync_copy(t_hbm, table_v)

    idx_bufs = (idx_v0, idx_v1)
    out_bufs = (out_v0, out_v1)
    idx_sems = (sem_i0, sem_i1)
    out_sems = (sem_o0, sem_o1)

    def idx_copy(c, buf, sem):
        return pltpu.make_async_copy(
            idx_hbm.at[pl.ds(c * _NSQ, _NSQ), pl.ds(bw, _BBLK)], buf, sem
        )

    def out_copy(c, buf, sem):
        return pltpu.make_async_copy(
            buf,
            out_hbm.at[pl.ds(0, _VOCAB), pl.ds(c * _NSQ, _NSQ),
                       pl.ds(bw, _BBLK)],
            sem,
        )

    def compute(idx_v, out_v):
        def bv_body(bv, _):
            o0 = bv * 16
            for si in range(_NSQ):
                rowids = idx_v[si, pl.ds(o0, 16)]
                fb = rowids * _TPAD
                for v in range(_VOCAB):
                    vals = plsc.load_gather(table_v, [fb + v])
                    out_v[v, si, pl.ds(o0, 16)] = vals
            return 0

        lax.fori_loop(0, _BBLK // 16, bv_body, 0)

    idx_copy(0, idx_bufs[0], idx_sems[0]).start()

    def pair_body(cc, _):
        for bsel in range(2):
            c = cc * 2 + bsel
            ivb, isem = idx_bufs[bsel], idx_sems[bsel]
            ovb, osem = out_bufs[bsel], out_sems[bsel]
            idx_copy(c, ivb, isem).wait()

            @pl.when(c + 1 < _N_CHUNKS)
            def _():
                idx_copy(c + 1, idx_bufs[1 - bsel], idx_sems[1 - bsel]).start()

            @pl.when(c >= 2)
            def _():
                out_copy(c, ovb, osem).wait()

            compute(ivb, ovb)
            out_copy(c, ovb, osem).start()
        return 0

    lax.fori_loop(0, _N_CHUNKS // 2, pair_body, 0)

    for bsel in range(2):
        c = _N_CHUNKS - 2 + bsel
        out_copy(c, out_bufs[bsel], out_sems[bsel]).wait()


def _sc_gather(table_flat, idx_t):
    mesh = plsc.VectorSubcoreMesh(core_axis_name="c", subcore_axis_name="s")
    kern = functools.partial(
        pl.kernel,
        mesh=mesh,
        compiler_params=pltpu.CompilerParams(needs_layout_passes=False),
        out_type=jax.ShapeDtypeStruct((_VOCAB, _SEQ, _BATCH), jnp.float32),
        scratch_types=[
            pltpu.VMEM((_VOCAB * _TPAD,), jnp.float32),
            pltpu.VMEM((_NSQ, _BBLK), jnp.int32),
            pltpu.VMEM((_NSQ, _BBLK), jnp.int32),
            pltpu.VMEM((_VOCAB, _NSQ, _BBLK), jnp.float32),
            pltpu.VMEM((_VOCAB, _NSQ, _BBLK), jnp.float32),
            pltpu.SemaphoreType.DMA,
            pltpu.SemaphoreType.DMA,
            pltpu.SemaphoreType.DMA,
            pltpu.SemaphoreType.DMA,
        ],
    )(_gather_body)
    return kern(table_flat, idx_t)


def kernel(indices, embed_table, W, b):
    table_flat, idx_t = _fuse_table(embed_table, W, b, indices)
    out_vsb = _sc_gather(table_flat, idx_t)
    return jnp.transpose(out_vsb, (2, 1, 0))


# speedup vs baseline: 4.1682x; 1.7948x over previous
import functools

import jax
import jax.numpy as jnp
from jax import lax
from jax.experimental import pallas as pl
from jax.experimental.pallas import tpu as pltpu
from jax.experimental.pallas import tpu_sc as plsc

_VOCAB = 50
_HIDDEN = 32
_BATCH = 4096
_SEQ = 200

_TPAD = 65
_NC = 2
_NS = 16
_NW = _NC * _NS
_BBLK = _BATCH // _NW
_NSQ = 4
_N_CHUNKS = _SEQ // _NSQ


def _fuse_body(e_ref, w_ref, b_ref, idx_ref, t_ref, it_ref):
    t_ref[...] = (
        jnp.dot(e_ref[...], w_ref[...], preferred_element_type=jnp.float32)
        + b_ref[...]
    )
    it_ref[...] = idx_ref[...].T


def _fuse_table(embed_table, W, b, indices):
    wp = jnp.zeros((_HIDDEN, _TPAD), jnp.float32).at[:, :_VOCAB].set(W)
    bp = jnp.zeros((1, _TPAD), jnp.float32).at[0, :_VOCAB].set(b)
    t, idx_t = pl.pallas_call(
        _fuse_body,
        out_shape=(
            jax.ShapeDtypeStruct((_VOCAB, _TPAD), jnp.float32),
            jax.ShapeDtypeStruct((_SEQ, _BATCH), jnp.int32),
        ),
    )(embed_table, wp, bp, indices)
    return t.reshape(_VOCAB * _TPAD), idx_t


def _gather_body(t_hbm, idx_hbm, out_hbm,
                 table_v, idx_v0, idx_v1, out_v0, out_v1,
                 sem_i0, sem_i1, sem_o0, sem_o1):
    wid = lax.axis_index("s") * _NC + lax.axis_index("c")
    bw = wid * _BBLK

    pltpu.sync_copy(t_hbm, table_v)

    idx_bufs = (idx_v0, idx_v1)
    out_bufs = (out_v0, out_v1)
    idx_sems = (sem_i0, sem_i1)
    out_sems = (sem_o0, sem_o1)

    def idx_copy(c, buf, sem):
        return pltpu.make_async_copy(
            idx_hbm.at[pl.ds(c * _NSQ, _NSQ), pl.ds(bw, _BBLK)], buf, sem
        )

    def out_copy(c, buf, sem):
        return pltpu.make_async_copy(
            buf,
            out_hbm.at[pl.ds(0, _VOCAB), pl.ds(c * _NSQ, _NSQ),
                       pl.ds(bw, _BBLK)],
            sem,
        )

    def compute(idx_v, out_v):
        @plsc.parallel_loop(0, _BBLK, 16)
        def bv_body(o0):
            for si in range(_NSQ):
                rowids = idx_v[si, pl.ds(o0, 16)]
                fb = rowids * _TPAD
                for v in range(_VOCAB):
                    vals = plsc.load_gather(table_v, [fb + v])
                    out_v[v, si, pl.ds(o0, 16)] = vals

    idx_copy(0, idx_bufs[0], idx_sems[0]).start()

    def pair_body(cc, _):
        for bsel in range(2):
            c = cc * 2 + bsel
            ivb, isem = idx_bufs[bsel], idx_sems[bsel]
            ovb, osem = out_bufs[bsel], out_sems[bsel]
            idx_copy(c, ivb, isem).wait()

            @pl.when(c + 1 < _N_CHUNKS)
            def _():
                idx_copy(c + 1, idx_bufs[1 - bsel], idx_sems[1 - bsel]).start()

            @pl.when(c >= 2)
            def _():
                out_copy(c, ovb, osem).wait()

            compute(ivb, ovb)
            out_copy(c, ovb, osem).start()
        return 0

    lax.fori_loop(0, _N_CHUNKS // 2, pair_body, 0)

    for bsel in range(2):
        c = _N_CHUNKS - 2 + bsel
        out_copy(c, out_bufs[bsel], out_sems[bsel]).wait()


def _sc_gather(table_flat, idx_t):
    mesh = plsc.VectorSubcoreMesh(core_axis_name="c", subcore_axis_name="s")
    kern = functools.partial(
        pl.kernel,
        mesh=mesh,
        compiler_params=pltpu.CompilerParams(needs_layout_passes=False),
        out_type=jax.ShapeDtypeStruct((_VOCAB, _SEQ, _BATCH), jnp.float32),
        scratch_types=[
            pltpu.VMEM((_VOCAB * _TPAD,), jnp.float32),
            pltpu.VMEM((_NSQ, _BBLK), jnp.int32),
            pltpu.VMEM((_NSQ, _BBLK), jnp.int32),
            pltpu.VMEM((_VOCAB, _NSQ, _BBLK), jnp.float32),
            pltpu.VMEM((_VOCAB, _NSQ, _BBLK), jnp.float32),
            pltpu.SemaphoreType.DMA,
            pltpu.SemaphoreType.DMA,
            pltpu.SemaphoreType.DMA,
            pltpu.SemaphoreType.DMA,
        ],
    )(_gather_body)
    return kern(table_flat, idx_t)


def kernel(indices, embed_table, W, b):
    table_flat, idx_t = _fuse_table(embed_table, W, b, indices)
    out_vsb = _sc_gather(table_flat, idx_t)
    return jnp.transpose(out_vsb, (2, 1, 0))
